# baseline (speedup 1.0000x reference)
import functools

import jax
import jax.numpy as jnp
from jax import lax
from jax.experimental import pallas as pl
from jax.experimental.pallas import tpu as pltpu
from jax.experimental.pallas import tpu_sc as plsc

_VOCAB = 1000000
_EMBED = 64
_BATCH = 16384
_HIST = 50
_NTOT = _BATCH * _HIST
_NHALF = _NTOT // 2
_NW = 32
_RPW = _NHALF // _NW
_GW = 128
_C = 512
_KG = _C // _GW
_NCH = _RPW // _C
_IDXROWS = _RPW // _GW
_STAGE = _IDXROWS + 4


def _sc_gather(idx_hbm, table_hbm, out_hbm, idx_v, rows_a, rows_b, gsem_a,
               gsem_b, osem_a, osem_b):
    wid = lax.axis_index("s") * 2 + lax.axis_index("c")
    base = wid * _RPW

    skew = (wid % 2) * 4
    a0 = pl.multiple_of(wid * _IDXROWS - skew, 8)
    pltpu.sync_copy(idx_hbm.at[pl.ds(a0, _STAGE)], idx_v)

    def fire_gathers(g, rows_v, sem):
        gm = lax.rem(g, _NCH)
        return [
            pltpu.async_copy(
                table_hbm.at[idx_v.at[skew + gm * _KG + j]],
                rows_v.at[pl.ds(j * _GW, _GW)],
                sem,
            )
            for j in range(_KG)
        ]

    def drain_gathers(rows_v, sem):
        for j in range(_KG):
            pltpu.make_async_copy(
                table_hbm.at[idx_v.at[j]],
                rows_v.at[pl.ds(j * _GW, _GW)],
                sem,
            ).wait()

    def store(g, rows_v, sem):
        return pltpu.async_copy(
            rows_v, out_hbm.at[pl.ds(pl.multiple_of(base + g * _C, _C), _C)],
            sem)

    def wait_store(g, rows_v, sem):
        pltpu.make_async_copy(
            rows_v, out_hbm.at[pl.ds(pl.multiple_of(base + g * _C, _C), _C)],
            sem).wait()

    fire_gathers(0, rows_a, gsem_a)
    drain_gathers(rows_a, gsem_a)
    fire_gathers(1, rows_b, gsem_b)
    store(0, rows_a, osem_a)

    def body(p, carry):
        g = 2 * p + 1
        drain_gathers(rows_b, gsem_b)
        wait_store(g - 1, rows_a, osem_a)
        fire_gathers(g + 1, rows_a, gsem_a)
        store(g, rows_b, osem_b)
        drain_gathers(rows_a, gsem_a)
        wait_store(g, rows_b, osem_b)
        fire_gathers(g + 2, rows_b, gsem_b)
        store(g + 1, rows_a, osem_a)
        return carry

    lax.fori_loop(0, (_NCH - 1) // 2, body, 0)

    drain_gathers(rows_b, gsem_b)
    wait_store(_NCH - 1, rows_a, osem_a)


_mesh = plsc.VectorSubcoreMesh(core_axis_name="c", subcore_axis_name="s")

_gather_call = functools.partial(
    pl.kernel,
    out_type=jax.ShapeDtypeStruct((_NHALF, _EMBED), jnp.float32),
    mesh=_mesh,
    compiler_params=pltpu.CompilerParams(use_tc_tiling_on_sc=False),
    scratch_types=[
        pltpu.VMEM((_STAGE, _GW), jnp.int32),
        pltpu.VMEM((_C, _EMBED), jnp.float32),
        pltpu.VMEM((_C, _EMBED), jnp.float32),
        pltpu.SemaphoreType.DMA,
        pltpu.SemaphoreType.DMA,
        pltpu.SemaphoreType.DMA,
        pltpu.SemaphoreType.DMA,
    ],
)(_sc_gather)


@jax.jit
def kernel(input, weight):
    idx = input.reshape(_NTOT // _GW, _GW).astype(jnp.int32)
    nh = _NHALF // _GW
    r1 = _gather_call(idx[:nh], weight)
    r2 = _gather_call(idx[nh:], weight)
    half = _BATCH // 2
    return jnp.concatenate(
        [r1.reshape(half, _HIST, _EMBED), r2.reshape(half, _HIST, _EMBED)],
        axis=0)
